# baseline (speedup 1.0000x reference)
import functools

import jax
import jax.numpy as jnp
from jax import lax
from jax.experimental import pallas as pl
from jax.experimental.pallas import tpu as pltpu
from jax.experimental.pallas import tpu_sc as plsc

N_STATES = 2
N_OBVS = 6
LANES = 16
NUM_CORES = 2
NUM_SUBCORES = 16
NUM_WORKERS = NUM_CORES * NUM_SUBCORES
CHUNK_ROWS = 40
NBUF = 2


def _make_sc_lookup(n_rows: int, n_cols: int):
    cols_per_worker = n_cols // NUM_WORKERS
    n_chunks = n_rows // CHUNK_ROWS
    assert cols_per_worker * NUM_WORKERS == n_cols
    assert cols_per_worker % LANES == 0
    assert n_chunks * CHUNK_ROWS == n_rows and n_chunks >= NBUF
    sh = (CHUNK_ROWS, cols_per_worker)

    mesh = plsc.VectorSubcoreMesh(core_axis_name="c", subcore_axis_name="s")

    @functools.partial(
        pl.kernel,
        mesh=mesh,
        out_type=jax.ShapeDtypeStruct((n_rows, n_cols), jnp.float32),
        compiler_params=pltpu.CompilerParams(needs_layout_passes=False),
        scratch_types=(
            [pltpu.VMEM(sh, jnp.int32) for _ in range(2 * NBUF)]
            + [pltpu.VMEM(sh, jnp.float32) for _ in range(NBUF)]
            + [pltpu.VMEM((LANES,), jnp.float32)]
            + [pltpu.SemaphoreType.DMA for _ in range(2 * NBUF)]
        ),
    )
    def sc_lookup(s_hbm, o_hbm, em_hbm, out_hbm, *scratch):
        s_v = scratch[0:NBUF]
        o_v = scratch[NBUF : 2 * NBUF]
        out_v = scratch[2 * NBUF : 3 * NBUF]
        em_v = scratch[3 * NBUF]
        sem_in = scratch[3 * NBUF + 1 : 3 * NBUF + 1 + NBUF]
        sem_out = scratch[3 * NBUF + 1 + NBUF :]

        wid = lax.axis_index("s") * NUM_CORES + lax.axis_index("c")
        cols = pl.ds(wid * cols_per_worker, cols_per_worker)

        def start_in(g):
            b = g % NBUF
            rows = pl.ds(g * CHUNK_ROWS, CHUNK_ROWS)
            return (
                pltpu.async_copy(s_hbm.at[rows, cols], s_v[b], sem_in[b]),
                pltpu.async_copy(o_hbm.at[rows, cols], o_v[b], sem_in[b]),
            )

        in_cp = {0: start_in(0)}
        pltpu.sync_copy(em_hbm, em_v)
        out_cp = {}
        for g in range(n_chunks):
            b = g % NBUF
            if g + 1 < n_chunks:
                in_cp[g + 1] = start_in(g + 1)
            for cp in in_cp.pop(g):
                cp.wait()
            if g >= NBUF:
                out_cp.pop(g - NBUF).wait()

            sb, ob, ub = s_v[b], o_v[b], out_v[b]
            spr = cols_per_worker // LANES

            @plsc.parallel_loop(0, CHUNK_ROWS * spr, 1, unroll=2)
            def body(t):
                r = t // spr
                c = (t % spr) * LANES
                flat = sb[r, pl.ds(c, LANES)] * N_OBVS + ob[r, pl.ds(c, LANES)]
                ub[r, pl.ds(c, LANES)] = plsc.load_gather(em_v, [flat])

            out_cp[g] = pltpu.async_copy(
                ub, out_hbm.at[pl.ds(g * CHUNK_ROWS, CHUNK_ROWS), cols], sem_out[b]
            )
        for g in sorted(out_cp):
            out_cp.pop(g).wait()

    return sc_lookup


def kernel(state, obs, log_em):
    n_rows, n_cols = state.shape
    em_pad = jnp.pad(log_em.reshape(N_STATES * N_OBVS), (0, LANES - N_STATES * N_OBVS))
    out_t = _make_sc_lookup(n_cols, n_rows)(state.T, obs.T, em_pad)
    return out_t.T


# speedup vs baseline: 1.0119x; 1.0119x over previous
import functools

import jax
import jax.numpy as jnp
from jax import lax
from jax.experimental import pallas as pl
from jax.experimental.pallas import tpu as pltpu
from jax.experimental.pallas import tpu_sc as plsc

N_STATES = 2
N_OBVS = 6
LANES = 16
NUM_CORES = 2
NUM_SUBCORES = 16
NUM_WORKERS = NUM_CORES * NUM_SUBCORES
CHUNK_ROWS = 40
NBUF = 2


def _make_sc_lookup(n_rows: int, n_cols: int):
    cols_per_worker = n_cols // NUM_WORKERS
    n_chunks = n_rows // CHUNK_ROWS
    assert cols_per_worker * NUM_WORKERS == n_cols
    assert cols_per_worker % LANES == 0
    assert n_chunks * CHUNK_ROWS == n_rows and n_chunks >= NBUF
    sh = (CHUNK_ROWS, cols_per_worker)

    mesh = plsc.VectorSubcoreMesh(core_axis_name="c", subcore_axis_name="s")

    @functools.partial(
        pl.kernel,
        mesh=mesh,
        out_type=jax.ShapeDtypeStruct((n_rows, n_cols), jnp.float32),
        compiler_params=pltpu.CompilerParams(needs_layout_passes=False),
        scratch_types=(
            [pltpu.VMEM(sh, jnp.int32) for _ in range(2 * NBUF)]
            + [pltpu.VMEM(sh, jnp.float32) for _ in range(NBUF)]
            + [pltpu.VMEM((LANES,), jnp.float32)]
            + [pltpu.SemaphoreType.DMA for _ in range(2 * NBUF)]
        ),
    )
    def sc_lookup(s_hbm, o_hbm, em_hbm, out_hbm, *scratch):
        s_v = scratch[0:NBUF]
        o_v = scratch[NBUF : 2 * NBUF]
        out_v = scratch[2 * NBUF : 3 * NBUF]
        em_v = scratch[3 * NBUF]
        sem_in = scratch[3 * NBUF + 1 : 3 * NBUF + 1 + NBUF]
        sem_out = scratch[3 * NBUF + 1 + NBUF :]

        wid = lax.axis_index("s") * NUM_CORES + lax.axis_index("c")
        cols = pl.ds(wid * cols_per_worker, cols_per_worker)

        def start_in(g):
            b = g % NBUF
            rows = pl.ds(g * CHUNK_ROWS, CHUNK_ROWS)
            return (
                pltpu.async_copy(s_hbm.at[rows, cols], s_v[b], sem_in[b]),
                pltpu.async_copy(o_hbm.at[rows, cols], o_v[b], sem_in[b]),
            )

        in_cp = {0: start_in(0)}
        pltpu.sync_copy(em_hbm, em_v)
        out_cp = {}
        for g in range(n_chunks):
            b = g % NBUF
            if g + 1 < n_chunks:
                in_cp[g + 1] = start_in(g + 1)
            for cp in in_cp.pop(g):
                cp.wait()
            if g >= NBUF:
                out_cp.pop(g - NBUF).wait()

            sb, ob, ub = s_v[b], o_v[b], out_v[b]
            spr = cols_per_worker // LANES

            @plsc.parallel_loop(0, CHUNK_ROWS * spr, 1, unroll=4)
            def body(t):
                r = t // spr
                c = (t % spr) * LANES
                flat = sb[r, pl.ds(c, LANES)] * N_OBVS + ob[r, pl.ds(c, LANES)]
                ub[r, pl.ds(c, LANES)] = plsc.load_gather(em_v, [flat])

            out_cp[g] = pltpu.async_copy(
                ub, out_hbm.at[pl.ds(g * CHUNK_ROWS, CHUNK_ROWS), cols], sem_out[b]
            )
        for g in sorted(out_cp):
            out_cp.pop(g).wait()

    return sc_lookup


def kernel(state, obs, log_em):
    n_rows, n_cols = state.shape
    em_pad = jnp.pad(log_em.reshape(N_STATES * N_OBVS), (0, LANES - N_STATES * N_OBVS))
    out_t = _make_sc_lookup(n_cols, n_rows)(state.T, obs.T, em_pad)
    return out_t.T


# speedup vs baseline: 1.0204x; 1.0084x over previous
import functools

import jax
import jax.numpy as jnp
from jax import lax
from jax.experimental import pallas as pl
from jax.experimental.pallas import tpu as pltpu
from jax.experimental.pallas import tpu_sc as plsc

N_STATES = 2
N_OBVS = 6
LANES = 16
NUM_CORES = 2
NUM_SUBCORES = 16
NUM_WORKERS = NUM_CORES * NUM_SUBCORES
CHUNK_ROWS = 40
NBUF = 2


def _make_sc_lookup(n_rows: int, n_cols: int):
    cols_per_worker = n_cols // NUM_WORKERS
    n_chunks = n_rows // CHUNK_ROWS
    assert cols_per_worker * NUM_WORKERS == n_cols
    assert cols_per_worker % LANES == 0
    assert n_chunks * CHUNK_ROWS == n_rows and n_chunks >= NBUF
    sh = (CHUNK_ROWS, cols_per_worker)

    mesh = plsc.VectorSubcoreMesh(core_axis_name="c", subcore_axis_name="s")

    @functools.partial(
        pl.kernel,
        mesh=mesh,
        out_type=jax.ShapeDtypeStruct((n_rows, n_cols), jnp.float32),
        compiler_params=pltpu.CompilerParams(needs_layout_passes=False),
        scratch_types=(
            [pltpu.VMEM(sh, jnp.int32) for _ in range(2 * NBUF)]
            + [pltpu.VMEM(sh, jnp.float32) for _ in range(NBUF)]
            + [pltpu.VMEM((LANES,), jnp.float32)]
            + [pltpu.VMEM((N_STATES, N_OBVS), jnp.float32)]
            + [pltpu.SemaphoreType.DMA for _ in range(2 * NBUF)]
        ),
    )
    def sc_lookup(s_hbm, o_hbm, em_hbm, out_hbm, *scratch):
        s_v = scratch[0:NBUF]
        o_v = scratch[NBUF : 2 * NBUF]
        out_v = scratch[2 * NBUF : 3 * NBUF]
        em_v = scratch[3 * NBUF]
        em2_v = scratch[3 * NBUF + 1]
        sem_in = scratch[3 * NBUF + 2 : 3 * NBUF + 2 + NBUF]
        sem_out = scratch[3 * NBUF + 2 + NBUF :]

        wid = lax.axis_index("s") * NUM_CORES + lax.axis_index("c")
        cols = pl.ds(wid * cols_per_worker, cols_per_worker)

        def start_in(g):
            b = g % NBUF
            rows = pl.ds(g * CHUNK_ROWS, CHUNK_ROWS)
            return (
                pltpu.async_copy(s_hbm.at[rows, cols], s_v[b], sem_in[b]),
                pltpu.async_copy(o_hbm.at[rows, cols], o_v[b], sem_in[b]),
            )

        in_cp = {0: start_in(0)}
        pltpu.sync_copy(em_hbm, em2_v)
        lane = lax.iota(jnp.int32, LANES)
        valid = lane < N_STATES * N_OBVS
        em_v[...] = plsc.load_gather(
            em2_v, [lane // N_OBVS, lane % N_OBVS], mask=valid
        )
        out_cp = {}
        for g in range(n_chunks):
            b = g % NBUF
            if g + 1 < n_chunks:
                in_cp[g + 1] = start_in(g + 1)
            for cp in in_cp.pop(g):
                cp.wait()
            if g >= NBUF:
                out_cp.pop(g - NBUF).wait()

            sb, ob, ub = s_v[b], o_v[b], out_v[b]
            spr = cols_per_worker // LANES

            @plsc.parallel_loop(0, CHUNK_ROWS * spr, 1, unroll=4)
            def body(t):
                r = t // spr
                c = (t % spr) * LANES
                flat = sb[r, pl.ds(c, LANES)] * N_OBVS + ob[r, pl.ds(c, LANES)]
                ub[r, pl.ds(c, LANES)] = plsc.load_gather(em_v, [flat])

            out_cp[g] = pltpu.async_copy(
                ub, out_hbm.at[pl.ds(g * CHUNK_ROWS, CHUNK_ROWS), cols], sem_out[b]
            )
        for g in sorted(out_cp):
            out_cp.pop(g).wait()

    return sc_lookup


def kernel(state, obs, log_em):
    n_rows, n_cols = state.shape
    out_t = _make_sc_lookup(n_cols, n_rows)(state.T, obs.T, log_em)
    return out_t.T
